# baseline (speedup 1.0000x reference)
import jax
import jax.numpy as jnp
from jax import lax
from jax.experimental import pallas as pl
from jax.experimental.pallas import tpu as pltpu
from jax.experimental.pallas import tpu_sc as plsc

N = 10000
E = 320000
D = 128
DH = D // 2
NC, NS, L = 2, 16, 16
CH = 128
CPT = 158
EPAD = NS * CPT * CH
NPAD = 10112
DUMMY = N
SEG = NPAD // NS
ZPAD = 640

_MESH = plsc.VectorSubcoreMesh(
    core_axis_name="c", subcore_axis_name="s", num_cores=NC, num_subcores=NS)


def _deg_body(dst_hbm, out_hbm, dstv, onesv, zerov, degsh):
    c = lax.axis_index("c")
    s = lax.axis_index("s")
    for g in range(CH // L):
        onesv[pl.ds(g * L, L)] = jnp.ones((L,), jnp.float32)
    for g in range(ZPAD // L):
        zerov[pl.ds(g * L, L)] = jnp.zeros((L,), jnp.float32)
    pltpu.sync_copy(zerov.at[pl.ds(0, SEG)], degsh.at[pl.ds(s * SEG, SEG)])
    plsc.subcore_barrier()
    pltpu.sync_copy(dst_hbm.at[s], dstv)

    def body(j, carry):
        pltpu.sync_copy(onesv, degsh.at[dstv.at[j]], add=True)
        return carry

    lax.fori_loop(c * (CPT // 2), (c + 1) * (CPT // 2), body, 0)

    plsc.subcore_barrier()
    wch = NPAD // NS + 8

    @pl.when(s < NS - 1)
    def _():
        pltpu.sync_copy(degsh.at[pl.ds(s * wch, wch)],
                        out_hbm.at[c, 0, pl.ds(s * wch, wch)])

    @pl.when(s == NS - 1)
    def _():
        last = NPAD - (NS - 1) * wch
        pltpu.sync_copy(degsh.at[pl.ds((NS - 1) * wch, last)],
                        out_hbm.at[c, 0, pl.ds((NS - 1) * wch, last)])


_deg_call = pl.kernel(
    _deg_body,
    out_type=jax.ShapeDtypeStruct((NC, 1, NPAD), jnp.float32),
    mesh=_MESH,
    scratch_types=[
        pltpu.VMEM((CPT, CH), jnp.int32),
        pltpu.VMEM((CH,), jnp.float32),
        pltpu.VMEM((ZPAD,), jnp.float32),
        pltpu.VMEM_SHARED((NPAD,), jnp.float32),
    ],
)


def _agg_body(p0_hbm, p1_hbm, src_hbm, dst_hbm, out_hbm, srcv, dstv, rows_a,
              rows_b, aggsh, gsem_a, gsem_b, ssem):
    c = lax.axis_index("c")
    s = lax.axis_index("s")

    def init(p_hbm):
        pltpu.sync_copy(p_hbm.at[pl.ds(s * SEG, SEG)],
                        aggsh.at[pl.ds(s * SEG, SEG)])

    @pl.when(c == 0)
    def _():
        init(p0_hbm)

    @pl.when(c == 1)
    def _():
        init(p1_hbm)

    plsc.subcore_barrier()
    pltpu.sync_copy(src_hbm.at[s], srcv)
    pltpu.sync_copy(dst_hbm.at[s], dstv)

    def run(p_hbm):
        bufs = ((rows_a, gsem_a), (rows_b, gsem_b))
        nbuf = len(bufs)
        for b, (buf, sem) in enumerate(bufs):
            pltpu.async_copy(p_hbm.at[srcv.at[b]], buf, sem)

        def body(g, carry):
            for b, (buf, sem) in enumerate(bufs):
                j = nbuf * g + b
                pltpu.make_async_copy(p_hbm.at[srcv.at[j]], buf, sem).wait()
                pltpu.async_copy(buf, aggsh.at[dstv.at[j]], ssem,
                                 add=True).wait()
                pltpu.async_copy(p_hbm.at[srcv.at[j + nbuf]], buf, sem)
            return carry

        lax.fori_loop(0, CPT // nbuf - 1, body, 0)
        for b, (buf, sem) in enumerate(bufs):
            j = CPT - nbuf + b
            pltpu.make_async_copy(p_hbm.at[srcv.at[j]], buf, sem).wait()
            pltpu.async_copy(buf, aggsh.at[dstv.at[j]], ssem, add=True).wait()

    @pl.when(c == 0)
    def _():
        run(p0_hbm)

    @pl.when(c == 1)
    def _():
        run(p1_hbm)

    plsc.subcore_barrier()
    pltpu.sync_copy(aggsh.at[pl.ds(s * SEG, SEG)],
                    out_hbm.at[c, pl.ds(s * SEG, SEG)])


_agg_call = pl.kernel(
    _agg_body,
    out_type=jax.ShapeDtypeStruct((NC, NPAD, DH), jnp.float32),
    mesh=_MESH,
    scratch_types=[
        pltpu.VMEM((CPT, CH), jnp.int32),
        pltpu.VMEM((CPT, CH), jnp.int32),
        pltpu.VMEM((CH, DH), jnp.float32),
        pltpu.VMEM((CH, DH), jnp.float32),
        pltpu.VMEM_SHARED((NPAD, DH), jnp.float32),
        pltpu.SemaphoreType.DMA,
        pltpu.SemaphoreType.DMA,
        pltpu.SemaphoreType.DMA,
    ],
    compiler_params=pltpu.CompilerParams(use_tc_tiling_on_sc=False),
)


BR = 632


def _dinv(deg_ref):
    return lax.rsqrt(deg_ref[:, 0:1] + deg_ref[:, 1:2] + 1.0)


def _split(h, out_ref):
    out_ref[0] = h[:, :DH]
    out_ref[1] = h[:, DH:]


def _mm1_body(x_ref, w_ref, deg_ref, out_ref):
    h = jnp.dot(x_ref[...], w_ref[...], preferred_element_type=jnp.float32)
    _split(h * _dinv(deg_ref), out_ref)


def _mid_body(agg_ref, deg_ref, b1_ref, g_ref, bb_ref, w2_ref, out_ref):
    dinv = _dinv(deg_ref)
    a = jnp.concatenate([agg_ref[0], agg_ref[1]], axis=-1)
    y = a * dinv + b1_ref[...]
    mu = jnp.mean(y, axis=-1, keepdims=True)
    var = jnp.mean((y - mu) ** 2, axis=-1, keepdims=True)
    z = (y - mu) * lax.rsqrt(var + 1e-5) * g_ref[...] + bb_ref[...]
    z = jnp.maximum(z, 0.0)
    h = jnp.dot(z, w2_ref[...], preferred_element_type=jnp.float32)
    _split(h * dinv, out_ref)


BRF = 400


def _fin_body(agg_ref, deg_ref, b2_ref, out_ref):
    a = jnp.concatenate([agg_ref[0], agg_ref[1]], axis=-1)
    out_ref[...] = a * _dinv(deg_ref) + b2_ref[...]


_SPLIT_SPEC = pl.BlockSpec((NC, BR, DH), lambda i: (0, i, 0))
_VEC_SPEC = pl.BlockSpec((1, D), lambda i: (0, 0))
_DEG_SPEC = pl.BlockSpec((BR, 2), lambda i: (i, 0))
_W_SPEC = pl.BlockSpec((D, D), lambda i: (0, 0))


def kernel(x, edge_index, W1, b1, W2, b2, ln_g, ln_b):
    src = edge_index[0].astype(jnp.int32)
    dst = edge_index[1].astype(jnp.int32)
    pad_e = EPAD - E
    pad_dst = DUMMY + (jnp.arange(pad_e, dtype=jnp.int32) % (NPAD - N))
    src_p = jnp.concatenate(
        [src, jnp.zeros((pad_e,), jnp.int32)]
    ).reshape(CPT, NS, CH).transpose(1, 0, 2)
    dst_p = jnp.concatenate(
        [dst, pad_dst]).reshape(CPT, NS, CH).transpose(1, 0, 2)
    b1r = b1.reshape(1, D)
    b2r = b2.reshape(1, D)
    gr = ln_g.reshape(1, D)
    bbr = ln_b.reshape(1, D)

    deg_pair = _deg_call(dst_p).reshape(NC, NPAD)
    deg_t = deg_pair.T

    p1 = pl.pallas_call(
        _mm1_body,
        grid=(N // BRF,),
        in_specs=[
            pl.BlockSpec((BRF, D), lambda i: (i, 0)),
            _W_SPEC,
            pl.BlockSpec((BRF, 2), lambda i: (i, 0)),
        ],
        out_specs=pl.BlockSpec((NC, BRF, DH), lambda i: (0, i, 0)),
        out_shape=jax.ShapeDtypeStruct((NC, NPAD, DH), jnp.float32),
    )(x, W1, deg_t)

    agg1 = _agg_call(p1[0], p1[1], src_p, dst_p)

    p2 = pl.pallas_call(
        _mid_body,
        grid=(NPAD // BR,),
        in_specs=[
            _SPLIT_SPEC,
            _DEG_SPEC,
            _VEC_SPEC,
            _VEC_SPEC,
            _VEC_SPEC,
            _W_SPEC,
        ],
        out_specs=_SPLIT_SPEC,
        out_shape=jax.ShapeDtypeStruct((NC, NPAD, DH), jnp.float32),
    )(agg1, deg_t, b1r, gr, bbr, W2)

    agg2 = _agg_call(p2[0], p2[1], src_p, dst_p)

    out = pl.pallas_call(
        _fin_body,
        grid=(N // BRF,),
        in_specs=[
            pl.BlockSpec((NC, BRF, DH), lambda i: (0, i, 0)),
            pl.BlockSpec((BRF, 2), lambda i: (i, 0)),
            _VEC_SPEC,
        ],
        out_specs=pl.BlockSpec((BRF, D), lambda i: (i, 0)),
        out_shape=jax.ShapeDtypeStruct((N, D), jnp.float32),
    )(agg2, deg_t, b2r)

    return out


# speedup vs baseline: 1.0088x; 1.0088x over previous
import jax
import jax.numpy as jnp
from jax import lax
from jax.experimental import pallas as pl
from jax.experimental.pallas import tpu as pltpu
from jax.experimental.pallas import tpu_sc as plsc

N = 10000
E = 320000
D = 128
DH = D // 2
NC, NS, L = 2, 16, 16
CH = 128
CPT = 158
EPAD = NS * CPT * CH
NPAD = 10112
DUMMY = N
SEG = NPAD // NS
ZPAD = 640

_MESH = plsc.VectorSubcoreMesh(
    core_axis_name="c", subcore_axis_name="s", num_cores=NC, num_subcores=NS)


def _deg_body(dst_hbm, out_hbm, dstv, onesv, zerov, degsh):
    c = lax.axis_index("c")
    s = lax.axis_index("s")
    for g in range(CH // L):
        onesv[pl.ds(g * L, L)] = jnp.ones((L,), jnp.float32)
    for g in range(ZPAD // L):
        zerov[pl.ds(g * L, L)] = jnp.zeros((L,), jnp.float32)
    pltpu.sync_copy(zerov.at[pl.ds(0, SEG)], degsh.at[pl.ds(s * SEG, SEG)])
    plsc.subcore_barrier()
    pltpu.sync_copy(dst_hbm.at[s], dstv)

    def body(j, carry):
        pltpu.sync_copy(onesv, degsh.at[dstv.at[j]], add=True)
        return carry

    lax.fori_loop(c * (CPT // 2), (c + 1) * (CPT // 2), body, 0)

    plsc.subcore_barrier()
    wch = NPAD // NS + 8

    @pl.when(s < NS - 1)
    def _():
        pltpu.sync_copy(degsh.at[pl.ds(s * wch, wch)],
                        out_hbm.at[c, 0, pl.ds(s * wch, wch)])

    @pl.when(s == NS - 1)
    def _():
        last = NPAD - (NS - 1) * wch
        pltpu.sync_copy(degsh.at[pl.ds((NS - 1) * wch, last)],
                        out_hbm.at[c, 0, pl.ds((NS - 1) * wch, last)])


_deg_call = pl.kernel(
    _deg_body,
    out_type=jax.ShapeDtypeStruct((NC, 1, NPAD), jnp.float32),
    mesh=_MESH,
    scratch_types=[
        pltpu.VMEM((CPT, CH), jnp.int32),
        pltpu.VMEM((CH,), jnp.float32),
        pltpu.VMEM((ZPAD,), jnp.float32),
        pltpu.VMEM_SHARED((NPAD,), jnp.float32),
    ],
)


def _agg_body(p0_hbm, p1_hbm, src_hbm, dst_hbm, out_hbm, srcv, dstv, rows_a,
              rows_b, aggsh, gsem_a, gsem_b, ssem):
    c = lax.axis_index("c")
    s = lax.axis_index("s")

    def init(p_hbm):
        pltpu.sync_copy(p_hbm.at[pl.ds(s * SEG, SEG)],
                        aggsh.at[pl.ds(s * SEG, SEG)])

    @pl.when(c == 0)
    def _():
        init(p0_hbm)

    @pl.when(c == 1)
    def _():
        init(p1_hbm)

    plsc.subcore_barrier()
    pltpu.sync_copy(src_hbm.at[s], srcv)
    pltpu.sync_copy(dst_hbm.at[s], dstv)

    def run(p_hbm):
        bufs = ((rows_a, gsem_a), (rows_b, gsem_b))
        nbuf = len(bufs)
        for b, (buf, sem) in enumerate(bufs):
            pltpu.async_copy(p_hbm.at[srcv.at[b]], buf, sem)

        def body(g, carry):
            for b, (buf, sem) in enumerate(bufs):
                j = nbuf * g + b
                pltpu.make_async_copy(p_hbm.at[srcv.at[j]], buf, sem).wait()
                pltpu.async_copy(buf, aggsh.at[dstv.at[j]], ssem,
                                 add=True).wait()
                pltpu.async_copy(p_hbm.at[srcv.at[j + nbuf]], buf, sem)
            return carry

        lax.fori_loop(0, CPT // nbuf - 1, body, 0)
        for b, (buf, sem) in enumerate(bufs):
            j = CPT - nbuf + b
            pltpu.make_async_copy(p_hbm.at[srcv.at[j]], buf, sem).wait()
            pltpu.async_copy(buf, aggsh.at[dstv.at[j]], ssem, add=True).wait()

    @pl.when(c == 0)
    def _():
        run(p0_hbm)

    @pl.when(c == 1)
    def _():
        run(p1_hbm)

    plsc.subcore_barrier()
    pltpu.sync_copy(aggsh.at[pl.ds(s * SEG, SEG)],
                    out_hbm.at[c, pl.ds(s * SEG, SEG)])


_agg_call = pl.kernel(
    _agg_body,
    out_type=jax.ShapeDtypeStruct((NC, NPAD, DH), jnp.float32),
    mesh=_MESH,
    scratch_types=[
        pltpu.VMEM((CPT, CH), jnp.int32),
        pltpu.VMEM((CPT, CH), jnp.int32),
        pltpu.VMEM((CH, DH), jnp.float32),
        pltpu.VMEM((CH, DH), jnp.float32),
        pltpu.VMEM_SHARED((NPAD, DH), jnp.float32),
        pltpu.SemaphoreType.DMA,
        pltpu.SemaphoreType.DMA,
        pltpu.SemaphoreType.DMA,
    ],
    compiler_params=pltpu.CompilerParams(use_tc_tiling_on_sc=False),
)


BR = 632


def _dinv(deg_ref):
    return lax.rsqrt(deg_ref[:, 0:1] + deg_ref[:, 1:2] + 1.0)


def _split(h, out_ref):
    out_ref[0] = h[:, :DH]
    out_ref[1] = h[:, DH:]


def _mm1_body(x_ref, w_ref, deg_ref, out_ref):
    h = jnp.dot(x_ref[...], w_ref[...], preferred_element_type=jnp.float32)
    _split(h * _dinv(deg_ref), out_ref)


def _mid_body(agg_ref, deg_ref, b1_ref, g_ref, bb_ref, w2_ref, out_ref):
    dinv = _dinv(deg_ref)
    a = jnp.concatenate([agg_ref[0], agg_ref[1]], axis=-1)
    y = a * dinv + b1_ref[...]
    mu = jnp.mean(y, axis=-1, keepdims=True)
    var = jnp.mean((y - mu) ** 2, axis=-1, keepdims=True)
    z = (y - mu) * lax.rsqrt(var + 1e-5) * g_ref[...] + bb_ref[...]
    z = jnp.maximum(z, 0.0)
    h = jnp.dot(z, w2_ref[...], preferred_element_type=jnp.float32)
    _split(h * dinv, out_ref)


BRF = 400


def _fin_body(agg_ref, deg_ref, b2_ref, out_ref):
    a = jnp.concatenate([agg_ref[0], agg_ref[1]], axis=-1)
    out_ref[...] = a * _dinv(deg_ref) + b2_ref[...]


_SPLIT_SPEC = pl.BlockSpec((NC, BR, DH), lambda i: (0, i, 0))
_VEC_SPEC = pl.BlockSpec((1, D), lambda i: (0, 0))
_DEG_SPEC = pl.BlockSpec((BR, 2), lambda i: (i, 0))
_W_SPEC = pl.BlockSpec((D, D), lambda i: (0, 0))


def kernel(x, edge_index, W1, b1, W2, b2, ln_g, ln_b):
    src = edge_index[0].astype(jnp.int32)
    dst = edge_index[1].astype(jnp.int32)
    pad_e = EPAD - E
    pad_dst = DUMMY + (jnp.arange(pad_e, dtype=jnp.int32) % (NPAD - N))
    src_p = jnp.concatenate(
        [src, jnp.zeros((pad_e,), jnp.int32)]
    ).reshape(CPT, NS, CH).transpose(1, 0, 2)
    dst_p = jnp.concatenate(
        [dst, pad_dst]).reshape(CPT, NS, CH).transpose(1, 0, 2)
    x_p = jnp.concatenate([x, jnp.zeros((NPAD - N, D), jnp.float32)])
    b1r = b1.reshape(1, D)
    b2r = b2.reshape(1, D)
    gr = ln_g.reshape(1, D)
    bbr = ln_b.reshape(1, D)

    deg_pair = _deg_call(dst_p).reshape(NC, NPAD)
    deg_t = deg_pair.T

    p1 = pl.pallas_call(
        _mm1_body,
        grid=(NPAD // BR,),
        in_specs=[
            pl.BlockSpec((BR, D), lambda i: (i, 0)),
            _W_SPEC,
            _DEG_SPEC,
        ],
        out_specs=_SPLIT_SPEC,
        out_shape=jax.ShapeDtypeStruct((NC, NPAD, DH), jnp.float32),
    )(x_p, W1, deg_t)

    agg1 = _agg_call(p1[0], p1[1], src_p, dst_p)

    p2 = pl.pallas_call(
        _mid_body,
        grid=(NPAD // BR,),
        in_specs=[
            _SPLIT_SPEC,
            _DEG_SPEC,
            _VEC_SPEC,
            _VEC_SPEC,
            _VEC_SPEC,
            _W_SPEC,
        ],
        out_specs=_SPLIT_SPEC,
        out_shape=jax.ShapeDtypeStruct((NC, NPAD, DH), jnp.float32),
    )(agg1, deg_t, b1r, gr, bbr, W2)

    agg2 = _agg_call(p2[0], p2[1], src_p, dst_p)

    out = pl.pallas_call(
        _fin_body,
        grid=(N // BRF,),
        in_specs=[
            pl.BlockSpec((NC, BRF, DH), lambda i: (0, i, 0)),
            pl.BlockSpec((BRF, 2), lambda i: (i, 0)),
            _VEC_SPEC,
        ],
        out_specs=pl.BlockSpec((BRF, D), lambda i: (i, 0)),
        out_shape=jax.ShapeDtypeStruct((N, D), jnp.float32),
    )(agg2, deg_t, b2r)

    return out
